# baseline (speedup 1.0000x reference)
import functools

import jax
import jax.numpy as jnp
from jax import lax
from jax.experimental import pallas as pl
from jax.experimental.pallas import tpu as pltpu
from jax.experimental.pallas import tpu_sc as plsc

_D = 64
_S = 50
_NB = 16384 // 128
_NW = 32
_BT_PER_W = _NB // _NW
_BLK = 128 * _S

_mesh = plsc.VectorSubcoreMesh(core_axis_name="c", subcore_axis_name="s")


@functools.partial(
    pl.kernel,
    mesh=_mesh,
    out_type=jax.ShapeDtypeStruct((_S, 8, _NB, 8, 128), jnp.float32),
    scratch_types=[
        pltpu.VMEM((_BLK,), jnp.int32),
        pltpu.VMEM((128,), jnp.int32),
        pltpu.VMEM((128, _D), jnp.float32),
        pltpu.VMEM((128, _D), jnp.float32),
        pltpu.VMEM((8, 8, 128), jnp.float32),
        pltpu.VMEM((8, 8, 128), jnp.float32),
        pltpu.SemaphoreType.DMA,
        pltpu.SemaphoreType.DMA,
        pltpu.SemaphoreType.DMA,
        pltpu.SemaphoreType.DMA,
    ],
    compiler_params=pltpu.CompilerParams(
        use_tc_tiling_on_sc=False, needs_layout_passes=False),
)
def _gather_kernel(idx_hbm, table_hbm, out_hbm, blk_v, slab_idx, rows0, rows1,
                   swz0, swz1, sg0, sg1, ss0, ss1):
    wid = lax.axis_index("s") * 2 + lax.axis_index("c")
    rows_v = (rows0, rows1)
    swz_v = (swz0, swz1)
    sem_g = (sg0, sg1)
    sem_s = (ss0, ss1)

    lane = lax.iota(jnp.int32, 16)
    lane50 = lane * 50
    row_ids = [lane + 16 * k for k in range(8)]

    def extract_idx(s):
        base = lane50 + jnp.full((16,), s, jnp.int32)
        for k in range(8):
            v = plsc.load_gather(blk_v, [base + (k * 16 * 50)])
            slab_idx[pl.ds(k * 16, 16)] = v

    def start_gather(b):
        pltpu.async_copy(table_hbm.at[slab_idx], rows_v[b], sem_g[b])

    def wait_gather(b):
        pltpu.make_async_copy(
            table_hbm.at[pl.ds(0, 128)], rows_v[b], sem_g[b]).wait()

    def transpose(b):
        @plsc.parallel_loop(0, 8, unroll=2)
        def _(dt):
            for di in range(8):
                col = jnp.full((16,), di, jnp.int32) + dt * 8
                vs = [plsc.load_gather(rows_v[b], [row_ids[k], col])
                      for k in range(8)]
                for k in range(8):
                    swz_v[b][dt, di, pl.ds(k * 16, 16)] = vs[k]

    def start_store(s, bt, b):
        pltpu.async_copy(swz_v[b], out_hbm.at[s, :, bt], sem_s[b])

    def wait_store(b):
        pltpu.make_async_copy(swz_v[b], out_hbm.at[0, :, 0], sem_s[b]).wait()

    def do_block(bt, carry):
        pltpu.sync_copy(idx_hbm.at[pl.ds(bt * _BLK, _BLK)], blk_v)

        extract_idx(0)
        start_gather(0)

        def body(i, c):
            s = 2 * i
            wait_gather(0)
            extract_idx(s + 1)
            start_gather(1)

            @pl.when(i > 0)
            def _():
                wait_store(0)

            transpose(0)
            start_store(s, bt, 0)

            wait_gather(1)

            @pl.when(i < _S // 2 - 1)
            def _():
                extract_idx(s + 2)
                start_gather(0)

            @pl.when(i > 0)
            def _():
                wait_store(1)

            transpose(1)
            start_store(s + 1, bt, 1)
            return c

        lax.fori_loop(0, _S // 2, body, 0)
        wait_store(0)
        wait_store(1)
        return carry

    lax.fori_loop(wid * _BT_PER_W, (wid + 1) * _BT_PER_W, do_block, 0)


def kernel(token_ids, emb):
    idx = token_ids.reshape(-1).astype(jnp.int32)
    out5 = _gather_kernel(idx, emb)
    return out5.transpose((2, 4, 0, 1, 3)).reshape(16384, _S, _D)
